# Initial kernel scaffold; baseline (speedup 1.0000x reference)
import jax
import jax.numpy as jnp
from jax.experimental import pallas as pl


def kernel(tokens, table):
    raise NotImplementedError("write your pallas kernel here")


# speedup vs baseline: 3.9586x; 3.9586x over previous
import functools

import jax
import jax.numpy as jnp
from jax import lax
from jax.experimental import pallas as pl
from jax.experimental.pallas import tpu as pltpu
from jax.experimental.pallas import tpu_sc as plsc

D = 64
SCALE = 8.0

NC = 2
NS = 16
NW = NC * NS

B = 4096 * 200
IDX_MINOR = 128
ROWS_PER_W = B // (NW * IDX_MINOR)
R = 2
CHUNK = R * IDX_MINOR
NCHUNK = ROWS_PER_W // R
NBUF = 4


def _scale_block(t_ref, o_ref):
    o_ref[...] = t_ref[...] * SCALE


def _scale_table(table):
    rows = table.shape[0]
    grid = 10
    blk = rows // grid
    return pl.pallas_call(
        _scale_block,
        grid=(grid,),
        in_specs=[pl.BlockSpec((blk, D), lambda i: (i, 0))],
        out_specs=pl.BlockSpec((blk, D), lambda i: (i, 0)),
        out_shape=jax.ShapeDtypeStruct((rows, D), jnp.float32),
    )(table)


@functools.partial(
    pl.kernel,
    out_type=jax.ShapeDtypeStruct((B, D), jnp.float32),
    mesh=plsc.VectorSubcoreMesh(core_axis_name="c", subcore_axis_name="s"),
    compiler_params=pltpu.CompilerParams(use_tc_tiling_on_sc=False),
    scratch_types=[
        pltpu.VMEM((ROWS_PER_W, IDX_MINOR), jnp.int32),
        pltpu.VMEM((CHUNK, D), jnp.float32),
        pltpu.VMEM((CHUNK, D), jnp.float32),
        pltpu.VMEM((CHUNK, D), jnp.float32),
        pltpu.VMEM((CHUNK, D), jnp.float32),
        pltpu.SemaphoreType.DMA,
        pltpu.SemaphoreType.DMA,
        pltpu.SemaphoreType.DMA,
        pltpu.SemaphoreType.DMA,
        pltpu.SemaphoreType.DMA,
        pltpu.SemaphoreType.DMA,
        pltpu.SemaphoreType.DMA,
        pltpu.SemaphoreType.DMA,
    ],
)
def _sc_gather(table_hbm, idx_hbm, out_hbm,
               idx_v, r0, r1, r2, r3, g0, g1, g2, g3, w0, w1, w2, w3):
    rows = (r0, r1, r2, r3)
    gsem = (g0, g1, g2, g3)
    wsem = (w0, w1, w2, w3)

    wid = lax.axis_index("s") * NC + lax.axis_index("c")
    rbase = wid * ROWS_PER_W
    obase = wid * (ROWS_PER_W * IDX_MINOR)

    pltpu.sync_copy(idx_hbm.at[pl.ds(rbase, ROWS_PER_W)], idx_v)

    def fire(c, b):
        for j in range(R):
            pltpu.async_copy(
                table_hbm.at[idx_v.at[c * R + j]],
                rows[b].at[pl.ds(j * IDX_MINOR, IDX_MINOR)],
                gsem[b])

    def drain(c, b):
        for j in range(R):
            pltpu.make_async_copy(
                table_hbm.at[idx_v.at[c * R + j]],
                rows[b].at[pl.ds(j * IDX_MINOR, IDX_MINOR)],
                gsem[b]).wait()

    def write(c, b):
        pltpu.async_copy(
            rows[b], out_hbm.at[pl.ds(obase + c * CHUNK, CHUNK)], wsem[b])

    def wait_write(c, b):
        pltpu.make_async_copy(
            rows[b], out_hbm.at[pl.ds(obase + c * CHUNK, CHUNK)], wsem[b]).wait()

    for c in range(NBUF - 1):
        fire(c, c)

    def step(p, carry):
        for b in range(NBUF):
            c = p * NBUF + b
            drain(c, b)
            write(c, b)
            fb = (b + NBUF - 1) % NBUF
            fc = c + NBUF - 1

            @pl.when(jnp.logical_and(fc >= NBUF, fc < NCHUNK))
            def _():
                wait_write(fc - NBUF, fb)

            @pl.when(fc < NCHUNK)
            def _():
                fire(fc, fb)
        return carry

    lax.fori_loop(0, NCHUNK // NBUF, step, 0)

    for b in range(NBUF):
        wait_write(NCHUNK - NBUF + b, b)


def kernel(tokens, table):
    idx = jnp.asarray(tokens, jnp.int32).reshape(NW * ROWS_PER_W, IDX_MINOR)
    scaled = _scale_table(jnp.asarray(table, jnp.float32))
    out = _sc_gather(scaled, idx)
    return out.reshape(tokens.shape + (D,))
